# baseline (speedup 1.0000x reference)
import functools

import jax
import jax.numpy as jnp
from jax.experimental import pallas as pl
from jax.experimental.pallas import tpu as pltpu


def _split3(x):
    h = x.astype(jnp.bfloat16).astype(jnp.float32)
    r = x - h
    m = r.astype(jnp.bfloat16).astype(jnp.float32)
    l = r - m
    return h, m, l


def _fill_s_aug(aug_ref, pts):
    n = pts.shape[0]
    sq = jnp.sum(pts * pts, axis=1, keepdims=True)
    h = sq.astype(jnp.bfloat16).astype(jnp.float32)
    l = sq - h
    aug_ref[:, 0:3] = -2.0 * pts
    aug_ref[:, 3:4] = h
    aug_ref[:, 4:5] = l
    aug_ref[:, 5:8] = jnp.ones((n, 3), jnp.float32)


def _fill_t_aug(aug_ref, pts):
    n = pts.shape[0]
    sq = jnp.sum(pts * pts, axis=1, keepdims=True)
    h, m, l = _split3(sq)
    aug_ref[:, 0:3] = pts
    aug_ref[:, 3:5] = jnp.ones((n, 2), jnp.float32)
    aug_ref[:, 5:6] = h
    aug_ref[:, 6:7] = m
    aug_ref[:, 7:8] = l


def _chamfer_kernel(
    s_ref,
    t_ref,
    fwd_ref,
    bwd_ref,
    saug_scr,
    taug_scr,
    srnd_scr,
    fmin_scr,
    bsum_scr,
    *,
    ch,
    vw,
):
    j = pl.program_id(1)
    nj = pl.num_programs(1)

    @pl.when(j == 0)
    def _():
        _fill_s_aug(saug_scr, s_ref[0])
        srnd_scr[...] = (
            (-2.0 * s_ref[0]).astype(jnp.bfloat16).astype(jnp.float32)
        )

    _fill_t_aug(taug_scr, t_ref[0])

    tm = taug_scr.shape[0]
    mxu_w = tm - vw
    saug = saug_scr[...]
    fmins = []
    bsums = []
    for c in range(mxu_w // ch):
        d2 = jax.lax.dot_general(
            saug,
            taug_scr[c * ch : (c + 1) * ch, :],
            (((1,), (1,)), ((), ())),
            preferred_element_type=jnp.float32,
        )
        fmins.append(jnp.min(d2, axis=1, keepdims=True))
        bsums.append(jnp.sum(jnp.min(d2, axis=0)))

    if vw:
        tv = taug_scr[mxu_w:tm, 0:3]
        tvr = tv.astype(jnp.bfloat16).astype(jnp.float32).T
        t_sq_row = (
            taug_scr[mxu_w:tm, 5:6] + taug_scr[mxu_w:tm, 6:7] + taug_scr[mxu_w:tm, 7:8]
        ).T
        s_sq_col = saug_scr[:, 3:4] + saug_scr[:, 4:5]
        srnd = srnd_scr[...]
        dot3 = (
            srnd[:, 0:1] * tvr[0:1, :]
            + srnd[:, 1:2] * tvr[1:2, :]
            + srnd[:, 2:3] * tvr[2:3, :]
        )
        d2v = (s_sq_col + t_sq_row) + dot3
        fmins.append(jnp.min(d2v, axis=1, keepdims=True))
        bsums.append(jnp.sum(jnp.min(d2v, axis=0)))

    tile_fmin = fmins[0]
    for fm in fmins[1:]:
        tile_fmin = jnp.minimum(tile_fmin, fm)
    tile_bsum = sum(bsums)

    @pl.when(j == 0)
    def _():
        fmin_scr[...] = tile_fmin
        bsum_scr[0] = tile_bsum

    @pl.when(j > 0)
    def _():
        fmin_scr[...] = jnp.minimum(fmin_scr[...], tile_fmin)
        bsum_scr[0] = bsum_scr[0] + tile_bsum

    @pl.when(j == nj - 1)
    def _():
        fwd_ref[...] = jnp.full(fwd_ref.shape, jnp.sum(fmin_scr[...]), jnp.float32)
        bwd_ref[...] = jnp.full(bwd_ref.shape, bsum_scr[0], jnp.float32)


@functools.partial(jax.jit, static_argnames=("tm", "ch", "vw"))
def _chamfer_sums(source_cloud, target_cloud, tm=4096, ch=512, vw=0):
    B, N, _ = source_cloud.shape
    M = target_cloud.shape[1]
    nj = M // tm

    fwd, bwd = pl.pallas_call(
        functools.partial(_chamfer_kernel, ch=ch, vw=vw),
        grid=(B, nj),
        in_specs=[
            pl.BlockSpec((1, N, 3), lambda b, j: (b, 0, 0)),
            pl.BlockSpec((1, tm, 3), lambda b, j: (b, j, 0)),
        ],
        out_specs=[
            pl.BlockSpec((1, 8, 128), lambda b, j: (b, 0, 0)),
            pl.BlockSpec((1, 8, 128), lambda b, j: (b, 0, 0)),
        ],
        out_shape=[
            jax.ShapeDtypeStruct((B, 8, 128), jnp.float32),
            jax.ShapeDtypeStruct((B, 8, 128), jnp.float32),
        ],
        scratch_shapes=[
            pltpu.VMEM((N, 8), jnp.float32),
            pltpu.VMEM((tm, 8), jnp.float32),
            pltpu.VMEM((N, 3), jnp.float32),
            pltpu.VMEM((N, 1), jnp.float32),
            pltpu.SMEM((1,), jnp.float32),
        ],
        compiler_params=pltpu.CompilerParams(
            dimension_semantics=("parallel", "arbitrary"),
            vmem_limit_bytes=100 * 1024 * 1024,
        ),
    )(source_cloud[:, :, :3], target_cloud[:, :, :3])
    return fwd[:, 0, 0], bwd[:, 0, 0]


def kernel(source_cloud, target_cloud):
    G = source_cloud.shape[1]
    fwd_sums, bwd_sums = _chamfer_sums(source_cloud, target_cloud)
    return (fwd_sums.mean() + bwd_sums.mean()) / G


# speedup vs baseline: 1.0034x; 1.0034x over previous
import functools

import jax
import jax.numpy as jnp
from jax.experimental import pallas as pl
from jax.experimental.pallas import tpu as pltpu


def _split3(x):
    h = x.astype(jnp.bfloat16).astype(jnp.float32)
    r = x - h
    m = r.astype(jnp.bfloat16).astype(jnp.float32)
    l = r - m
    return h, m, l


def _fill_s_aug(aug_ref, pts):
    n = pts.shape[0]
    sq = jnp.sum(pts * pts, axis=1, keepdims=True)
    h = sq.astype(jnp.bfloat16).astype(jnp.float32)
    l = sq - h
    aug_ref[:, 0:3] = -2.0 * pts
    aug_ref[:, 3:4] = h
    aug_ref[:, 4:5] = l
    aug_ref[:, 5:8] = jnp.ones((n, 3), jnp.float32)


def _fill_t_aug(aug_ref, pts):
    n = pts.shape[0]
    sq = jnp.sum(pts * pts, axis=1, keepdims=True)
    h, m, l = _split3(sq)
    aug_ref[:, 0:3] = pts
    aug_ref[:, 3:5] = jnp.ones((n, 2), jnp.float32)
    aug_ref[:, 5:6] = h
    aug_ref[:, 6:7] = m
    aug_ref[:, 7:8] = l


def _chamfer_kernel(
    s_ref, t_ref, fwd_ref, bwd_ref, saug_scr, taug_scr, fmin_scr, bsum_scr, *, ch
):
    j = pl.program_id(1)
    nj = pl.num_programs(1)

    @pl.when(j == 0)
    def _():
        _fill_s_aug(saug_scr, s_ref[0])

    _fill_t_aug(taug_scr, t_ref[0])

    tm = taug_scr.shape[0]
    saug = saug_scr[...]
    fmins = []
    bsums = []
    for c in range(tm // ch):
        d2 = jax.lax.dot_general(
            saug,
            taug_scr[c * ch : (c + 1) * ch, :],
            (((1,), (1,)), ((), ())),
            preferred_element_type=jnp.float32,
        )
        fmins.append(jnp.min(d2, axis=1, keepdims=True))
        bsums.append(jnp.sum(jnp.min(d2, axis=0)))

    tile_fmin = fmins[0]
    for fm in fmins[1:]:
        tile_fmin = jnp.minimum(tile_fmin, fm)
    tile_bsum = sum(bsums)

    @pl.when(j == 0)
    def _():
        fmin_scr[...] = tile_fmin
        bsum_scr[0] = tile_bsum

    @pl.when(j > 0)
    def _():
        fmin_scr[...] = jnp.minimum(fmin_scr[...], tile_fmin)
        bsum_scr[0] = bsum_scr[0] + tile_bsum

    @pl.when(j == nj - 1)
    def _():
        fwd_ref[...] = jnp.full(fwd_ref.shape, jnp.sum(fmin_scr[...]), jnp.float32)
        bwd_ref[...] = jnp.full(bwd_ref.shape, bsum_scr[0], jnp.float32)


@functools.partial(jax.jit, static_argnames=("tm", "ch"))
def _chamfer_sums(source_cloud, target_cloud, tm=4096, ch=2048):
    B, N, _ = source_cloud.shape
    M = target_cloud.shape[1]
    nj = M // tm

    fwd, bwd = pl.pallas_call(
        functools.partial(_chamfer_kernel, ch=ch),
        grid=(B, nj),
        in_specs=[
            pl.BlockSpec((1, N, 3), lambda b, j: (b, 0, 0)),
            pl.BlockSpec((1, tm, 3), lambda b, j: (b, j, 0)),
        ],
        out_specs=[
            pl.BlockSpec((1, 8, 128), lambda b, j: (b, 0, 0)),
            pl.BlockSpec((1, 8, 128), lambda b, j: (b, 0, 0)),
        ],
        out_shape=[
            jax.ShapeDtypeStruct((B, 8, 128), jnp.float32),
            jax.ShapeDtypeStruct((B, 8, 128), jnp.float32),
        ],
        scratch_shapes=[
            pltpu.VMEM((N, 8), jnp.float32),
            pltpu.VMEM((tm, 8), jnp.float32),
            pltpu.VMEM((N, 1), jnp.float32),
            pltpu.SMEM((1,), jnp.float32),
        ],
        compiler_params=pltpu.CompilerParams(
            dimension_semantics=("parallel", "arbitrary"),
            vmem_limit_bytes=100 * 1024 * 1024,
        ),
    )(source_cloud[:, :, :3], target_cloud[:, :, :3])
    return fwd[:, 0, 0], bwd[:, 0, 0]


def kernel(source_cloud, target_cloud):
    G = source_cloud.shape[1]
    fwd_sums, bwd_sums = _chamfer_sums(source_cloud, target_cloud)
    return (fwd_sums.mean() + bwd_sums.mean()) / G
